# Initial kernel scaffold; baseline (speedup 1.0000x reference)
import jax
import jax.numpy as jnp
from jax.experimental import pallas as pl


def kernel(x, fw_adjs, init_rel, Ws, W_loops, W_rels, biases, loop_rels):
    raise NotImplementedError("write your pallas kernel here")


# speedup vs baseline: 1540.7741x; 1540.7741x over previous
import jax
import jax.numpy as jnp
from jax.experimental import pallas as pl


def _compgcn_kernel(adj_ref, x_ref, rel_ref, ws_ref, wl_ref, wr_ref,
                    b_ref, lr_ref, out_ref):
    n = x_ref.shape[0]
    r_count = adj_ref.shape[0]
    num_layers = ws_ref.shape[0]
    f32 = jnp.float32
    dn_t = (((0,), (0,)), ((), ()))
    dn = (((1,), (0,)), ((), ()))
    prec = jax.lax.Precision.HIGHEST

    ones_col = jnp.ones((n, 1), f32)
    deg = jnp.zeros((n, 1), f32)
    masks = []
    for et in range(r_count):
        m = (adj_ref[et] > 0.5).astype(f32)
        masks.append(m)
        deg = deg + jax.lax.dot_general(m, ones_col, dn_t, precision=prec)
    norm = jnp.where(deg > 0.0, jax.lax.rsqrt(deg), 0.0)

    h = x_ref[...]
    r = rel_ref[...]
    for l in range(num_layers):
        hn = h * norm
        p = jnp.zeros_like(h)
        for et in range(r_count):
            comp = hn * r[et:et + 1, :]
            p = p + jax.lax.dot_general(masks[et], comp, dn_t,
                                        precision=prec)
        agg = jax.lax.dot_general(p, ws_ref[l], dn, precision=prec) * norm
        loop = jax.lax.dot_general(h * lr_ref[l], wl_ref[l], dn,
                                   precision=prec)
        h = jnp.tanh(agg + loop + b_ref[l:l + 1, :])
        if l + 1 < num_layers:
            r = jax.lax.dot_general(r, wr_ref[l], dn, precision=prec)
    out_ref[...] = h


@jax.jit
def kernel(x, fw_adjs, init_rel, Ws, W_loops, W_rels, biases, loop_rels):
    n, h_dim = x.shape
    r_count = fw_adjs.shape[0]
    rel = init_rel[:r_count]
    return pl.pallas_call(
        _compgcn_kernel,
        out_shape=jax.ShapeDtypeStruct((n, h_dim), x.dtype),
    )(fw_adjs, x, rel, Ws, W_loops, W_rels, biases, loop_rels)


# speedup vs baseline: 3629.1734x; 2.3554x over previous
import jax
import jax.numpy as jnp
from jax.experimental import pallas as pl


def _compgcn_kernel(adj_ref, x_ref, rel_ref, ws_ref, wl_ref, wr_ref,
                    b_ref, lr_ref, out_ref):
    n = x_ref.shape[0]
    r_count = adj_ref.shape[0]
    num_layers = ws_ref.shape[0]
    f32 = jnp.float32
    dn_t = (((0,), (0,)), ((), ()))
    dn = (((1,), (0,)), ((), ()))
    prec = jax.lax.Precision.DEFAULT

    ones_col = jnp.ones((n, 1), f32)
    deg = jnp.zeros((n, 1), f32)
    masks = []
    for et in range(r_count):
        m = (adj_ref[et] > 0.5).astype(f32)
        masks.append(m)
        deg = deg + jax.lax.dot_general(m, ones_col, dn_t, precision=prec)
    norm = jnp.where(deg > 0.0, jax.lax.rsqrt(deg), 0.0)

    h = x_ref[...]
    r = rel_ref[...]
    for l in range(num_layers):
        hn = h * norm
        p = jnp.zeros_like(h)
        for et in range(r_count):
            comp = hn * r[et:et + 1, :]
            p = p + jax.lax.dot_general(masks[et], comp, dn_t,
                                        precision=prec)
        agg = jax.lax.dot_general(p, ws_ref[l], dn, precision=prec) * norm
        loop = jax.lax.dot_general(h * lr_ref[l], wl_ref[l], dn,
                                   precision=prec)
        h = jnp.tanh(agg + loop + b_ref[l:l + 1, :])
        if l + 1 < num_layers:
            r = jax.lax.dot_general(r, wr_ref[l], dn, precision=prec)
    out_ref[...] = h


@jax.jit
def kernel(x, fw_adjs, init_rel, Ws, W_loops, W_rels, biases, loop_rels):
    n, h_dim = x.shape
    r_count = fw_adjs.shape[0]
    rel = init_rel[:r_count]
    return pl.pallas_call(
        _compgcn_kernel,
        out_shape=jax.ShapeDtypeStruct((n, h_dim), x.dtype),
    )(fw_adjs, x, rel, Ws, W_loops, W_rels, biases, loop_rels)
